# Initial kernel scaffold; baseline (speedup 1.0000x reference)
import jax
import jax.numpy as jnp
from jax.experimental import pallas as pl


def kernel(user_ids, item_ids, user_emb, item_emb, user_bias, item_bias):
    raise NotImplementedError("write your pallas kernel here")


# speedup vs baseline: 1.5404x; 1.5404x over previous
import jax
import jax.numpy as jnp
from jax.experimental import pallas as pl
from jax.experimental.pallas import tpu as pltpu

_TILE = 8192
_UNROLL = 32


def _gather_dot_kernel(uid_ref, iid_ref, u_tab_ref, i_tab_ref, out_ref,
                       p_ref):
    def chunk(ci, carry):
        base = ci * _UNROLL
        for j in range(_UNROLL):
            m = base + j
            p_ref[m] = u_tab_ref[uid_ref[0, m]] * i_tab_ref[iid_ref[0, m]]
        return carry

    jax.lax.fori_loop(0, _TILE // _UNROLL, chunk, 0)

    p = p_ref[...].reshape(_TILE, 128).astype(jnp.bfloat16)
    ones = jnp.ones((8, 128), jnp.bfloat16)
    acc = jax.lax.dot_general(ones, p, (((1,), (1,)), ((), ())),
                              preferred_element_type=jnp.float32)
    out_ref[...] = acc[0:1, :]


def kernel(user_ids, item_ids, user_emb, item_emb, user_bias, item_bias):
    B = user_ids.shape[0]
    Nu, D = user_emb.shape
    Ni = item_emb.shape[0]
    Daug = D + 2

    u_aug = jnp.concatenate(
        [user_emb.astype(jnp.float32),
         user_bias.reshape(Nu, 1).astype(jnp.float32),
         jnp.ones((Nu, 1), jnp.float32)], axis=1)
    i_aug = jnp.concatenate(
        [item_emb.astype(jnp.float32),
         jnp.ones((Ni, 1), jnp.float32),
         item_bias.reshape(Ni, 1).astype(jnp.float32)], axis=1)

    Dpad = ((Daug + 127) // 128) * 128
    if Dpad != Daug:
        u_aug = jnp.pad(u_aug, ((0, 0), (0, Dpad - Daug)))
        i_aug = jnp.pad(i_aug, ((0, 0), (0, Dpad - Daug)))
    u_tab = u_aug.reshape(Nu, 1, Dpad)
    i_tab = i_aug.reshape(Ni, 1, Dpad)

    Bp = pl.cdiv(B, _TILE) * _TILE
    pad = Bp - B
    uid = jnp.pad(user_ids.astype(jnp.int32), (0, pad)).reshape(1, Bp)
    iid = jnp.pad(item_ids.astype(jnp.int32), (0, pad)).reshape(1, Bp)

    out = pl.pallas_call(
        _gather_dot_kernel,
        out_shape=jax.ShapeDtypeStruct((1, Bp), jnp.float32),
        grid=(Bp // _TILE,),
        in_specs=[
            pl.BlockSpec((1, _TILE), lambda i: (0, i),
                         memory_space=pltpu.SMEM),
            pl.BlockSpec((1, _TILE), lambda i: (0, i),
                         memory_space=pltpu.SMEM),
            pl.BlockSpec((Nu, 1, Dpad), lambda i: (0, 0, 0)),
            pl.BlockSpec((Ni, 1, Dpad), lambda i: (0, 0, 0)),
        ],
        out_specs=pl.BlockSpec((1, _TILE), lambda i: (0, i)),
        scratch_shapes=[pltpu.VMEM((_TILE, 1, 128), jnp.float32)],
        compiler_params=pltpu.CompilerParams(
            dimension_semantics=("parallel",),
            vmem_limit_bytes=48 * 1024 * 1024),
    )(uid, iid, u_tab, i_tab)

    return out[0, :B]
